# baseline (speedup 1.0000x reference)
import functools

import jax
import jax.numpy as jnp
from jax.experimental import pallas as pl

NV = 51
NP = 56


def _leaky(h):
    return jnp.where(h >= 0, h, 0.01 * h)


def _edge_head(x, ws, bs, scales, shifts, depth):
    a = jnp.abs(x[:, None, :] - x[None, :, :]).reshape(NP * NP, x.shape[1])
    h = a
    for l in range(depth):
        h = jnp.dot(h, ws[l][...], preferred_element_type=jnp.float32) + bs[l][0]
        if l < depth - 1:
            h = _leaky(h * scales[l][0] + shifts[l][0])
    return h


def _stats_kernel(*args, depth):
    nw = depth
    x_ref = args[0]
    ws = args[1:1 + nw]
    bs = args[1 + nw:1 + 2 * nw]
    sc = args[1 + 2 * nw:1 + 2 * nw + (depth - 1)]
    sh = args[1 + 2 * nw + (depth - 1):1 + 2 * nw + 2 * (depth - 1)]
    sum_ref, sq_ref = args[-2], args[-1]
    h = _edge_head(x_ref[0], ws, bs, sc, sh, depth)
    r = jax.lax.broadcasted_iota(jnp.int32, (NP * NP, 1), 0)
    n = r // NP
    m = r - n * NP
    mask = ((n < NV) & (m < NV)).astype(jnp.float32)
    hm = h * mask
    sum_ref[0, 0, :] = jnp.sum(hm, axis=0)
    sq_ref[0, 0, :] = jnp.sum(hm * h, axis=0)


def _final_kernel(*args):
    x_ref = args[0]
    ws = args[1:6]
    bs = args[6:11]
    sc = args[11:15]
    sh = args[15:19]
    wg1, wg2, bg = args[19], args[20], args[21]
    gout_ref, gsum_ref, gsq_ref = args[-3], args[-2], args[-1]
    x = x_ref[0]
    h5 = _edge_head(x, ws, bs, sc, sh, 5).reshape(NP, NP)
    ii = jax.lax.broadcasted_iota(jnp.int32, (NP, NP), 0)
    jj = jax.lax.broadcasted_iota(jnp.int32, (NP, NP), 1)
    le = h5 + jnp.where(ii == jj, -1e8, 0.0) + jnp.where(jj >= NV, -1e9, 0.0)
    mx = jnp.max(le, axis=1, keepdims=True)
    e = jnp.exp(le - mx)
    w = e / jnp.sum(e, axis=1, keepdims=True)
    y = jnp.dot(w, x, preferred_element_type=jnp.float32)
    gout = (jnp.dot(x, wg1[...], preferred_element_type=jnp.float32)
            + jnp.dot(y, wg2[...], preferred_element_type=jnp.float32) + bg[0])
    ni = jax.lax.broadcasted_iota(jnp.int32, (NP, 1), 0)
    vmask = (ni < NV).astype(jnp.float32)
    gout = gout * vmask
    gout_ref[0] = gout
    gsum_ref[0, 0, :] = jnp.sum(gout, axis=0)
    gsq_ref[0, 0, :] = jnp.sum(gout * gout, axis=0)


def _final0_kernel(*args):
    x_ref = args[0]
    ws = args[1:6]
    bs = args[6:11]
    sc = args[11:15]
    sh = args[15:19]
    wg1, wg2, bg = args[19], args[20], args[21]
    logit_ref, sig_ref = args[-2], args[-1]
    x = x_ref[0]
    a = jnp.abs(x[0:1, :] - x)
    h = a
    for l in range(5):
        h = jnp.dot(h, ws[l][...], preferred_element_type=jnp.float32) + bs[l][0]
        if l < 4:
            h = _leaky(h * sc[l][0] + sh[l][0])
    ri = jax.lax.broadcasted_iota(jnp.int32, (NP, 1), 0)
    le = h + jnp.where(ri == 0, -1e8, 0.0) + jnp.where(ri >= NV, -1e9, 0.0)
    mx = jnp.max(le, axis=0, keepdims=True)
    e = jnp.exp(le - mx)
    w = e / jnp.sum(e, axis=0, keepdims=True)
    y = jax.lax.dot_general(w, x, (((0,), (0,)), ((), ())),
                            preferred_element_type=jnp.float32)
    gl = (jnp.dot(x[0:1, :], wg1[...], preferred_element_type=jnp.float32)
          + jnp.dot(y, wg2[...], preferred_element_type=jnp.float32) + bg[0])
    logit_ref[0, 0, :] = gl[0]
    sig_ref[0, 0, :] = (1.0 / (1.0 + jnp.exp(-gl)))[0]


def _bn_act_kernel(g_ref, scale_ref, shift_ref, out_ref):
    ni = jax.lax.broadcasted_iota(jnp.int32, (NP, 1), 0)
    vmask = (ni < NV).astype(jnp.float32)
    out_ref[0] = _leaky(g_ref[0] * scale_ref[0] + shift_ref[0]) * vmask


def _full_spec(shape):
    nd = len(shape)
    return pl.BlockSpec(shape, lambda b: (0,) * nd)


def _run_stats(xp, ws, bs, scales, shifts, depth):
    B, _, F = xp.shape
    C = ws[depth - 1].shape[1]
    ins = [xp] + list(ws[:depth]) + list(bs[:depth]) + \
        list(scales[:depth - 1]) + list(shifts[:depth - 1])
    in_specs = [pl.BlockSpec((1, NP, F), lambda b: (b, 0, 0))] + \
        [_full_spec(a.shape) for a in ins[1:]]
    out_shape = [jax.ShapeDtypeStruct((B, 1, C), jnp.float32)] * 2
    out_specs = [pl.BlockSpec((1, 1, C), lambda b: (b, 0, 0))] * 2
    s, sq = pl.pallas_call(
        functools.partial(_stats_kernel, depth=depth),
        grid=(B,), in_specs=in_specs, out_specs=out_specs,
        out_shape=out_shape)(*ins)
    return jnp.sum(s[:, 0, :], axis=0), jnp.sum(sq[:, 0, :], axis=0)


def _finalize(s, sq, cnt, g, beta):
    mean = s / cnt
    var = sq / cnt - mean * mean
    scale = (g * jax.lax.rsqrt(var + 1e-5)).reshape(1, -1)
    shift = (beta - mean * scale[0]).reshape(1, -1)
    return scale, shift


def _wcompute_gconv(xp, wc, gcp, last):
    B, _, F = xp.shape
    ws = wc["w"]
    bs = [b.reshape(1, -1) for b in wc["b"]]
    cnt = float(B * NV * NV)
    scales, shifts = [], []
    for d in range(1, 5):
        s, sq = _run_stats(xp, ws, bs, scales, shifts, d)
        sc, sh = _finalize(s, sq, cnt, wc["g"][d - 1], wc["beta"][d - 1])
        scales.append(sc)
        shifts.append(sh)
    wg1, wg2 = gcp["w"][:F], gcp["w"][F:]
    bg = gcp["b"].reshape(1, -1)
    Fo = wg1.shape[1]
    ins = [xp] + list(ws) + bs + scales + shifts + [wg1, wg2, bg]
    in_specs = [pl.BlockSpec((1, NP, F), lambda b: (b, 0, 0))] + \
        [_full_spec(a.shape) for a in ins[1:]]
    if last:
        out_shape = [jax.ShapeDtypeStruct((B, 1, Fo), jnp.float32)] * 2
        out_specs = [pl.BlockSpec((1, 1, Fo), lambda b: (b, 0, 0))] * 2
        logits, sig = pl.pallas_call(
            _final0_kernel, grid=(B,), in_specs=in_specs,
            out_specs=out_specs, out_shape=out_shape)(*ins)
        return logits[:, 0, :], sig[:, 0, :]
    out_shape = [jax.ShapeDtypeStruct((B, NP, Fo), jnp.float32),
                 jax.ShapeDtypeStruct((B, 1, Fo), jnp.float32),
                 jax.ShapeDtypeStruct((B, 1, Fo), jnp.float32)]
    out_specs = [pl.BlockSpec((1, NP, Fo), lambda b: (b, 0, 0)),
                 pl.BlockSpec((1, 1, Fo), lambda b: (b, 0, 0)),
                 pl.BlockSpec((1, 1, Fo), lambda b: (b, 0, 0))]
    gout, gs, gq = pl.pallas_call(
        _final_kernel, grid=(B,), in_specs=in_specs,
        out_specs=out_specs, out_shape=out_shape)(*ins)
    gs = jnp.sum(gs[:, 0, :], axis=0)
    gq = jnp.sum(gq[:, 0, :], axis=0)
    gscale, gshift = _finalize(gs, gq, float(B * NV), gcp["g"], gcp["beta"])
    act = pl.pallas_call(
        _bn_act_kernel, grid=(B,),
        in_specs=[pl.BlockSpec((1, NP, Fo), lambda b: (b, 0, 0)),
                  _full_spec(gscale.shape), _full_spec(gshift.shape)],
        out_specs=pl.BlockSpec((1, NP, Fo), lambda b: (b, 0, 0)),
        out_shape=jax.ShapeDtypeStruct((B, NP, Fo), jnp.float32))(
            gout, gscale, gshift)
    return act


def kernel(z, zi_s, labels_yi, params):
    B = z.shape[0]
    zero_pad = jnp.zeros((1, B, labels_yi.shape[2]), dtype=labels_yi.dtype)
    lab_all = jnp.concatenate([zero_pad, labels_yi], axis=0)
    z_all = jnp.concatenate([z[None], zi_s], axis=0)
    nodes = jnp.transpose(jnp.concatenate([z_all, lab_all], axis=2), (1, 0, 2))
    xp = jnp.pad(nodes, ((0, 0), (0, NP - NV), (0, 0)))
    for i in range(2):
        act = _wcompute_gconv(xp, params["wc"][i], params["gc"][i], last=False)
        xp = jnp.concatenate([xp, act], axis=2)
    logits, sig = _wcompute_gconv(xp, params["wc"][2], params["gc"][2],
                                  last=True)
    return (sig, logits)
